# Initial kernel scaffold; baseline (speedup 1.0000x reference)
import jax
import jax.numpy as jnp
from jax.experimental import pallas as pl


def kernel(x, batch, q_star, W_ih, W_hh, b_ih, b_hh):
    raise NotImplementedError("write your pallas kernel here")


# speedup vs baseline: 6.0065x; 6.0065x over previous
import functools

import jax
import jax.numpy as jnp
from jax.experimental import pallas as pl
from jax.experimental.pallas import tpu as pltpu

_NEG = -1e30


def _kern(batch_ref, x_ref, qs0_ref, wih_ref, whh_ref, b_ref,
          out_ref, qs, h, c, m, d, raccT, *, nsteps, nblk, bsz, dim):
    s = pl.program_id(0)
    j = pl.program_id(1)

    @pl.when(j == 0)
    def _step_head():
        @pl.when(s == 0)
        def _seed():
            qs[...] = qs0_ref[...]

        h_prev = jnp.where(s == 0, jnp.zeros_like(h[...]), h[...])
        c_prev = jnp.where(s == 0, jnp.zeros_like(c[...]), c[...])
        gates = (
            jax.lax.dot_general(qs[...], wih_ref[...], (((1,), (1,)), ((), ())),
                                preferred_element_type=jnp.float32,
                                precision=jax.lax.Precision.HIGHEST)
            + jax.lax.dot_general(h_prev, whh_ref[...], (((1,), (1,)), ((), ())),
                                  preferred_element_type=jnp.float32,
                                  precision=jax.lax.Precision.HIGHEST)
            + b_ref[...]
        )
        gi = jax.nn.sigmoid(gates[:, 0 * dim:1 * dim])
        gf = jax.nn.sigmoid(gates[:, 1 * dim:2 * dim])
        gg = jnp.tanh(gates[:, 2 * dim:3 * dim])
        go = jax.nn.sigmoid(gates[:, 3 * dim:4 * dim])
        c_new = gf * c_prev + gi * gg
        h[...] = go * jnp.tanh(c_new)
        c[...] = c_new
        m[...] = jnp.full_like(m[...], _NEG)
        d[...] = jnp.zeros_like(d[...])
        raccT[...] = jnp.zeros_like(raccT[...])

    q = h[...]
    xb = x_ref[...]
    bcol = batch_ref[0]
    oh = (bcol == jax.lax.broadcasted_iota(jnp.int32, (xb.shape[0], bsz), 1)
          ).astype(jnp.float32)

    E = jax.lax.dot_general(xb, q, (((1,), (1,)), ((), ())),
                            preferred_element_type=jnp.float32,
                            precision=jax.lax.Precision.HIGHEST)
    e = jnp.sum(E * oh, axis=1, keepdims=True)

    m_old = m[...]
    blk_max = jnp.max(jnp.where(oh > 0, E, _NEG), axis=0, keepdims=True)
    m_new = jnp.maximum(m_old, blk_max)
    scale = jnp.exp(m_old - m_new)
    mg = jnp.sum(oh * m_new, axis=1, keepdims=True)
    w = jnp.exp(e - mg)
    m[...] = m_new
    d_new = d[...] * scale + jnp.sum(oh * w, axis=0, keepdims=True)
    d[...] = d_new
    wx = w * xb
    raccT_new = raccT[...] * scale + jax.lax.dot_general(
        wx, oh, (((0,), (0,)), ((), ())),
        preferred_element_type=jnp.float32,
        precision=jax.lax.Precision.HIGHEST)
    raccT[...] = raccT_new

    @pl.when(j == nblk - 1)
    def _step_tail():
        rT = raccT_new / (d_new + 1e-16)
        r = rT.T
        qs[...] = jnp.concatenate([q, r], axis=1)

        @pl.when(s == nsteps - 1)
        def _emit():
            out_ref[...] = jnp.concatenate([q, r], axis=1)


def kernel(x, batch, q_star, W_ih, W_hh, b_ih, b_hh):
    n, dim = x.shape
    bsz, two_d = q_star.shape
    nsteps = 3
    blk = next(r for r in (2000, 1000, 400, 200, 80, 40, 16, 8, 1)
               if n % r == 0 and (r % 8 == 0 or r == 1))
    nblk = n // blk

    batch3 = batch.astype(jnp.int32).reshape(nblk, blk, 1)
    bias = (b_ih + b_hh).reshape(1, 4 * dim).astype(jnp.float32)

    grid = (nsteps, nblk)
    out = pl.pallas_call(
        functools.partial(_kern, nsteps=nsteps, nblk=nblk, bsz=bsz, dim=dim),
        grid=grid,
        in_specs=[
            pl.BlockSpec((1, blk, 1), lambda s, j: (j, 0, 0)),
            pl.BlockSpec((blk, dim), lambda s, j: (j, 0)),
            pl.BlockSpec((bsz, two_d), lambda s, j: (0, 0)),
            pl.BlockSpec(W_ih.shape, lambda s, j: (0, 0)),
            pl.BlockSpec(W_hh.shape, lambda s, j: (0, 0)),
            pl.BlockSpec((1, 4 * dim), lambda s, j: (0, 0)),
        ],
        out_specs=pl.BlockSpec((bsz, two_d), lambda s, j: (0, 0)),
        out_shape=jax.ShapeDtypeStruct((bsz, two_d), jnp.float32),
        scratch_shapes=[
            pltpu.VMEM((bsz, two_d), jnp.float32),
            pltpu.VMEM((bsz, dim), jnp.float32),
            pltpu.VMEM((bsz, dim), jnp.float32),
            pltpu.VMEM((1, bsz), jnp.float32),
            pltpu.VMEM((1, bsz), jnp.float32),
            pltpu.VMEM((dim, bsz), jnp.float32),
        ],
        compiler_params=pltpu.CompilerParams(
            dimension_semantics=("arbitrary", "arbitrary")),
    )(batch3, x, q_star, W_ih, W_hh, bias)
    return out


# speedup vs baseline: 13.5735x; 2.2598x over previous
import functools

import jax
import jax.numpy as jnp
from jax.experimental import pallas as pl
from jax.experimental.pallas import tpu as pltpu

_NEG = -1e30


def _split_bf16(a):
    hi = a.astype(jnp.bfloat16)
    lo = (a - hi.astype(jnp.float32)).astype(jnp.bfloat16)
    return hi, lo


def _dot_bf16(a, b, dims):
    return jax.lax.dot_general(a, b, (dims, ((), ())),
                               preferred_element_type=jnp.float32)


def _kern(batch_ref, xh_ref, xl_ref, qs0_ref, wih_ref, whh_ref, b_ref,
          out_ref, qs, h, c, m, d, raccT, *, nsteps, nblk, bsz, dim):
    s = pl.program_id(0)
    j = pl.program_id(1)

    @pl.when(j == 0)
    def _step_head():
        @pl.when(s == 0)
        def _seed():
            qs[...] = qs0_ref[...]

        h_prev = jnp.where(s == 0, jnp.zeros_like(h[...]), h[...])
        c_prev = jnp.where(s == 0, jnp.zeros_like(c[...]), c[...])
        gates = (
            jax.lax.dot_general(qs[...], wih_ref[...], (((1,), (1,)), ((), ())),
                                preferred_element_type=jnp.float32,
                                precision=jax.lax.Precision.HIGHEST)
            + jax.lax.dot_general(h_prev, whh_ref[...], (((1,), (1,)), ((), ())),
                                  preferred_element_type=jnp.float32,
                                  precision=jax.lax.Precision.HIGHEST)
            + b_ref[...]
        )
        gi = jax.nn.sigmoid(gates[:, 0 * dim:1 * dim])
        gf = jax.nn.sigmoid(gates[:, 1 * dim:2 * dim])
        gg = jnp.tanh(gates[:, 2 * dim:3 * dim])
        go = jax.nn.sigmoid(gates[:, 3 * dim:4 * dim])
        c_new = gf * c_prev + gi * gg
        h[...] = go * jnp.tanh(c_new)
        c[...] = c_new
        m[...] = jnp.full_like(m[...], _NEG)
        d[...] = jnp.zeros_like(d[...])
        raccT[...] = jnp.zeros_like(raccT[...])

    q = h[...]
    xh = xh_ref[...]
    xl = xl_ref[...]
    bcol = batch_ref[0]
    oh = (bcol == jax.lax.broadcasted_iota(jnp.int32, (xh.shape[0], bsz), 1)
          ).astype(jnp.bfloat16)

    qh, ql = _split_bf16(q)
    cdims = ((1,), (1,))
    E = (_dot_bf16(xh, qh, cdims) + _dot_bf16(xh, ql, cdims)
         + _dot_bf16(xl, qh, cdims))

    m_old = m[...]
    blk_max = jnp.max(jnp.where(oh > 0, E, _NEG), axis=0, keepdims=True)
    m_new = jnp.maximum(m_old, blk_max)
    scale = jnp.exp(m_old - m_new)
    w = jnp.exp(jnp.sum(oh * (E - m_new), axis=1, keepdims=True))
    m[...] = m_new
    d_new = d[...] * scale + jnp.sum(oh * w, axis=0, keepdims=True)
    d[...] = d_new
    wh = w.astype(jnp.bfloat16)
    raccT_new = (raccT[...] * scale
                 + _dot_bf16(wh * xh, oh, ((0,), (0,))))
    raccT[...] = raccT_new

    @pl.when(j == nblk - 1)
    def _step_tail():
        rT = raccT_new / (d_new + 1e-16)
        r = rT.T
        qs[...] = jnp.concatenate([q, r], axis=1)

        @pl.when(s == nsteps - 1)
        def _emit():
            out_ref[...] = jnp.concatenate([q, r], axis=1)


def kernel(x, batch, q_star, W_ih, W_hh, b_ih, b_hh):
    n, dim = x.shape
    bsz, two_d = q_star.shape
    nsteps = 3
    blk = next(r for r in (2000, 1000, 400, 200, 80, 40, 16, 8, 1)
               if n % r == 0 and (r % 8 == 0 or r == 1))
    nblk = n // blk

    batch3 = batch.astype(jnp.int32).reshape(nblk, blk, 1)
    bias = (b_ih + b_hh).reshape(1, 4 * dim).astype(jnp.float32)
    x_hi = x.astype(jnp.bfloat16)
    x_lo = (x - x_hi.astype(jnp.float32)).astype(jnp.bfloat16)

    grid = (nsteps, nblk)
    out = pl.pallas_call(
        functools.partial(_kern, nsteps=nsteps, nblk=nblk, bsz=bsz, dim=dim),
        grid=grid,
        in_specs=[
            pl.BlockSpec((1, blk, 1), lambda s, j: (j, 0, 0)),
            pl.BlockSpec((blk, dim), lambda s, j: (j, 0)),
            pl.BlockSpec((blk, dim), lambda s, j: (j, 0)),
            pl.BlockSpec((bsz, two_d), lambda s, j: (0, 0)),
            pl.BlockSpec(W_ih.shape, lambda s, j: (0, 0)),
            pl.BlockSpec(W_hh.shape, lambda s, j: (0, 0)),
            pl.BlockSpec((1, 4 * dim), lambda s, j: (0, 0)),
        ],
        out_specs=pl.BlockSpec((bsz, two_d), lambda s, j: (0, 0)),
        out_shape=jax.ShapeDtypeStruct((bsz, two_d), jnp.float32),
        scratch_shapes=[
            pltpu.VMEM((bsz, two_d), jnp.float32),
            pltpu.VMEM((bsz, dim), jnp.float32),
            pltpu.VMEM((bsz, dim), jnp.float32),
            pltpu.VMEM((1, bsz), jnp.float32),
            pltpu.VMEM((1, bsz), jnp.float32),
            pltpu.VMEM((dim, bsz), jnp.float32),
        ],
        compiler_params=pltpu.CompilerParams(
            dimension_semantics=("arbitrary", "arbitrary")),
    )(batch3, x_hi, x_lo, q_star, W_ih, W_hh, bias)
    return out


# speedup vs baseline: 14.5364x; 1.0709x over previous
import functools

import jax
import jax.numpy as jnp
from jax.experimental import pallas as pl
from jax.experimental.pallas import tpu as pltpu

_NEG = -1e30


def _split_bf16(a):
    hi = a.astype(jnp.bfloat16)
    lo = (a - hi.astype(jnp.float32)).astype(jnp.bfloat16)
    return hi, lo


def _dot_bf16(a, b, dims):
    return jax.lax.dot_general(a, b, (dims, ((), ())),
                               preferred_element_type=jnp.float32)


def _kern(batch_ref, xh_ref, xl_ref, qs0_ref, wih_ref, whh_ref, b_ref,
          out_ref, qs, h, c, m, d, raccT, *, nsteps, nblk, bsz, dim):
    s = pl.program_id(0)
    j = pl.program_id(1)

    @pl.when(j == 0)
    def _step_head():
        @pl.when(s == 0)
        def _seed():
            qs[...] = qs0_ref[...]

        h_prev = jnp.where(s == 0, jnp.zeros_like(h[...]), h[...])
        c_prev = jnp.where(s == 0, jnp.zeros_like(c[...]), c[...])
        gates = (
            jax.lax.dot_general(qs[...], wih_ref[...], (((1,), (1,)), ((), ())),
                                preferred_element_type=jnp.float32,
                                precision=jax.lax.Precision.HIGHEST)
            + jax.lax.dot_general(h_prev, whh_ref[...], (((1,), (1,)), ((), ())),
                                  preferred_element_type=jnp.float32,
                                  precision=jax.lax.Precision.HIGHEST)
            + b_ref[...]
        )
        gi = jax.nn.sigmoid(gates[:, 0 * dim:1 * dim])
        gf = jax.nn.sigmoid(gates[:, 1 * dim:2 * dim])
        gg = jnp.tanh(gates[:, 2 * dim:3 * dim])
        go = jax.nn.sigmoid(gates[:, 3 * dim:4 * dim])
        c_new = gf * c_prev + gi * gg
        h[...] = go * jnp.tanh(c_new)
        c[...] = c_new
        m[...] = jnp.full_like(m[...], _NEG)
        d[...] = jnp.zeros_like(d[...])
        raccT[...] = jnp.zeros_like(raccT[...])

    q = h[...]
    xh = xh_ref[...]
    xl = xl_ref[...]
    bcol = batch_ref[0]
    oh = (bcol == jax.lax.broadcasted_iota(jnp.int32, (xh.shape[0], bsz), 1)
          ).astype(jnp.bfloat16)

    qh, ql = _split_bf16(q)
    cdims = ((1,), (1,))
    E = (_dot_bf16(xh, qh, cdims) + _dot_bf16(xh, ql, cdims)
         + _dot_bf16(xl, qh, cdims))

    m_old = m[...]
    blk_max = jnp.max(jnp.where(oh > 0, E, _NEG), axis=0, keepdims=True)
    m_new = jnp.maximum(m_old, blk_max)
    scale = jnp.exp(m_old - m_new)
    w = jnp.exp(jnp.sum(oh * (E - m_new), axis=1, keepdims=True))
    m[...] = m_new
    d_new = d[...] * scale + jnp.sum(oh * w, axis=0, keepdims=True)
    d[...] = d_new
    wh = w.astype(jnp.bfloat16)
    raccT_new = (raccT[...] * scale
                 + _dot_bf16(wh * xh, oh, ((0,), (0,))))
    raccT[...] = raccT_new

    @pl.when(j == nblk - 1)
    def _step_tail():
        rT = raccT_new / (d_new + 1e-16)
        r = rT.T
        qs[...] = jnp.concatenate([q, r], axis=1)

        @pl.when(s == nsteps - 1)
        def _emit():
            out_ref[...] = jnp.concatenate([q, r], axis=1)


def kernel(x, batch, q_star, W_ih, W_hh, b_ih, b_hh):
    n, dim = x.shape
    bsz, two_d = q_star.shape
    nsteps = 3
    blk = next(r for r in (5000, 2000, 1000, 400, 200, 80, 40, 16, 8, 1)
               if n % r == 0 and (r % 8 == 0 or r == 1))
    nblk = n // blk

    batch3 = batch.astype(jnp.int32).reshape(nblk, blk, 1)
    bias = (b_ih + b_hh).reshape(1, 4 * dim).astype(jnp.float32)
    x_hi = x.astype(jnp.bfloat16)
    x_lo = (x - x_hi.astype(jnp.float32)).astype(jnp.bfloat16)

    grid = (nsteps, nblk)
    out = pl.pallas_call(
        functools.partial(_kern, nsteps=nsteps, nblk=nblk, bsz=bsz, dim=dim),
        grid=grid,
        in_specs=[
            pl.BlockSpec((1, blk, 1), lambda s, j: (j, 0, 0)),
            pl.BlockSpec((blk, dim), lambda s, j: (j, 0)),
            pl.BlockSpec((blk, dim), lambda s, j: (j, 0)),
            pl.BlockSpec((bsz, two_d), lambda s, j: (0, 0)),
            pl.BlockSpec(W_ih.shape, lambda s, j: (0, 0)),
            pl.BlockSpec(W_hh.shape, lambda s, j: (0, 0)),
            pl.BlockSpec((1, 4 * dim), lambda s, j: (0, 0)),
        ],
        out_specs=pl.BlockSpec((bsz, two_d), lambda s, j: (0, 0)),
        out_shape=jax.ShapeDtypeStruct((bsz, two_d), jnp.float32),
        scratch_shapes=[
            pltpu.VMEM((bsz, two_d), jnp.float32),
            pltpu.VMEM((bsz, dim), jnp.float32),
            pltpu.VMEM((bsz, dim), jnp.float32),
            pltpu.VMEM((1, bsz), jnp.float32),
            pltpu.VMEM((1, bsz), jnp.float32),
            pltpu.VMEM((dim, bsz), jnp.float32),
        ],
        compiler_params=pltpu.CompilerParams(
            dimension_semantics=("arbitrary", "arbitrary")),
    )(batch3, x_hi, x_lo, q_star, W_ih, W_hh, bias)
    return out


# speedup vs baseline: 15.7826x; 1.0857x over previous
import functools

import jax
import jax.numpy as jnp
from jax.experimental import pallas as pl
from jax.experimental.pallas import tpu as pltpu

_NEG = -1e30


def _split_bf16(a):
    hi = a.astype(jnp.bfloat16)
    lo = (a - hi.astype(jnp.float32)).astype(jnp.bfloat16)
    return hi, lo


def _dot_bf16(a, b, dims):
    return jax.lax.dot_general(a, b, (dims, ((), ())),
                               preferred_element_type=jnp.float32)


def _kern(batch_ref, xh_ref, xl_ref, qs0_ref, wih_ref, whh_ref, b_ref,
          out_ref, qs, h, c, m, d, raccT, *, nsteps, nblk, bsz, dim):
    s = pl.program_id(0)
    j = pl.program_id(1)

    @pl.when(j == 0)
    def _step_head():
        @pl.when(s == 0)
        def _seed():
            qs[...] = qs0_ref[...]

        h_prev = jnp.where(s == 0, jnp.zeros_like(h[...]), h[...])
        c_prev = jnp.where(s == 0, jnp.zeros_like(c[...]), c[...])
        gates = (
            jax.lax.dot_general(qs[...], wih_ref[...], (((1,), (1,)), ((), ())),
                                preferred_element_type=jnp.float32,
                                precision=jax.lax.Precision.HIGHEST)
            + jax.lax.dot_general(h_prev, whh_ref[...], (((1,), (1,)), ((), ())),
                                  preferred_element_type=jnp.float32,
                                  precision=jax.lax.Precision.HIGHEST)
            + b_ref[...]
        )
        gi = jax.nn.sigmoid(gates[:, 0 * dim:1 * dim])
        gf = jax.nn.sigmoid(gates[:, 1 * dim:2 * dim])
        gg = jnp.tanh(gates[:, 2 * dim:3 * dim])
        go = jax.nn.sigmoid(gates[:, 3 * dim:4 * dim])
        c_new = gf * c_prev + gi * gg
        h[...] = go * jnp.tanh(c_new)
        c[...] = c_new
        m[...] = jnp.full_like(m[...], _NEG)
        d[...] = jnp.zeros_like(d[...])
        raccT[...] = jnp.zeros_like(raccT[...])

    q = h[...]
    xh = xh_ref[...]
    xl = xl_ref[...]
    bcol = batch_ref[0]
    oh = (bcol == jax.lax.broadcasted_iota(jnp.int32, (xh.shape[0], bsz), 1)
          ).astype(jnp.bfloat16)

    qh = q.astype(jnp.bfloat16)
    cdims = ((1,), (1,))
    E = _dot_bf16(xh, qh, cdims) + _dot_bf16(xl, qh, cdims)

    m_old = m[...]
    blk_max = jnp.max(jnp.where(oh > 0, E, _NEG), axis=0, keepdims=True)
    m_new = jnp.maximum(m_old, blk_max)
    scale = jnp.exp(m_old - m_new)
    w = jnp.exp(jnp.sum(oh * (E - m_new), axis=1, keepdims=True))
    m[...] = m_new
    d_new = d[...] * scale + jnp.sum(oh * w, axis=0, keepdims=True)
    d[...] = d_new
    wh = w.astype(jnp.bfloat16)
    raccT_new = (raccT[...] * scale
                 + _dot_bf16(wh * xh, oh, ((0,), (0,))))
    raccT[...] = raccT_new

    @pl.when(j == nblk - 1)
    def _step_tail():
        rT = raccT_new / (d_new + 1e-16)
        r = rT.T
        qs[...] = jnp.concatenate([q, r], axis=1)

        @pl.when(s == nsteps - 1)
        def _emit():
            out_ref[...] = jnp.concatenate([q, r], axis=1)


def kernel(x, batch, q_star, W_ih, W_hh, b_ih, b_hh):
    n, dim = x.shape
    bsz, two_d = q_star.shape
    nsteps = 3
    blk = next(r for r in (5000, 2000, 1000, 400, 200, 80, 40, 16, 8, 1)
               if n % r == 0 and (r % 8 == 0 or r == 1))
    nblk = n // blk

    batch3 = batch.astype(jnp.int32).reshape(nblk, blk, 1)
    bias = (b_ih + b_hh).reshape(1, 4 * dim).astype(jnp.float32)
    x_hi = x.astype(jnp.bfloat16)
    x_lo = (x - x_hi.astype(jnp.float32)).astype(jnp.bfloat16)

    grid = (nsteps, nblk)
    out = pl.pallas_call(
        functools.partial(_kern, nsteps=nsteps, nblk=nblk, bsz=bsz, dim=dim),
        grid=grid,
        in_specs=[
            pl.BlockSpec((1, blk, 1), lambda s, j: (j, 0, 0)),
            pl.BlockSpec((blk, dim), lambda s, j: (j, 0)),
            pl.BlockSpec((blk, dim), lambda s, j: (j, 0)),
            pl.BlockSpec((bsz, two_d), lambda s, j: (0, 0)),
            pl.BlockSpec(W_ih.shape, lambda s, j: (0, 0)),
            pl.BlockSpec(W_hh.shape, lambda s, j: (0, 0)),
            pl.BlockSpec((1, 4 * dim), lambda s, j: (0, 0)),
        ],
        out_specs=pl.BlockSpec((bsz, two_d), lambda s, j: (0, 0)),
        out_shape=jax.ShapeDtypeStruct((bsz, two_d), jnp.float32),
        scratch_shapes=[
            pltpu.VMEM((bsz, two_d), jnp.float32),
            pltpu.VMEM((bsz, dim), jnp.float32),
            pltpu.VMEM((bsz, dim), jnp.float32),
            pltpu.VMEM((1, bsz), jnp.float32),
            pltpu.VMEM((1, bsz), jnp.float32),
            pltpu.VMEM((dim, bsz), jnp.float32),
        ],
        compiler_params=pltpu.CompilerParams(
            dimension_semantics=("arbitrary", "arbitrary")),
    )(batch3, x_hi, x_lo, q_star, W_ih, W_hh, bias)
    return out
